# Initial kernel scaffold; baseline (speedup 1.0000x reference)
import jax
import jax.numpy as jnp
from jax.experimental import pallas as pl


def kernel(x, x_cov, edge_index, adj_vals, gamma, beta, W1a, b1a, W2a, b2a, W3a, W1b, b1b, W2b, b2b, W3b, Wm1, bm1, Wm2, bm2, Wm3, bm3, prelu_a):
    raise NotImplementedError("write your pallas kernel here")


# speedup vs baseline: 5.7175x; 5.7175x over previous
import functools

import jax
import jax.numpy as jnp
from jax import lax
from jax.experimental import pallas as pl
from jax.experimental.pallas import tpu as pltpu
from jax.experimental.pallas import tpu_sc as plsc

N_NODES = 10000
FDIM = 128
N_EDGES = 160000

NC = 2
NS = 16
NW = NC * NS
CHUNK = 128
EPT = 5120
E_PAD = NW * EPT
NCHUNK = EPT // CHUNK
N_PAD = 10240
RPT = N_PAD // NS
VECS_PER_ROW = FDIM // 16

_mesh = plsc.VectorSubcoreMesh(core_axis_name="c", subcore_axis_name="s")
_sc_params = pltpu.CompilerParams(needs_layout_passes=False)


@functools.partial(
    pl.kernel, mesh=_mesh, compiler_params=_sc_params,
    out_type=[jax.ShapeDtypeStruct((NW, N_PAD), jnp.float32),
              jax.ShapeDtypeStruct((NW, N_PAD), jnp.float32)],
    scratch_types=[pltpu.VMEM((EPT,), jnp.int32),
                   pltpu.VMEM((EPT,), jnp.int32),
                   pltpu.VMEM((N_PAD,), jnp.float32),
                   pltpu.VMEM((N_PAD,), jnp.float32)],
)
def _deg_kernel(src_hbm, dst_hbm, do_hbm, di_hbm, src_v, dst_v, do_v, di_v):
    c = lax.axis_index("c")
    s = lax.axis_index("s")
    w = s * NC + c
    base = w * EPT
    pltpu.sync_copy(src_hbm.at[pl.ds(base, EPT)], src_v)
    pltpu.sync_copy(dst_hbm.at[pl.ds(base, EPT)], dst_v)
    zv = jnp.zeros((16,), jnp.float32)

    def zbody(i, carry):
        do_v[pl.ds(i * 16, 16)] = zv
        di_v[pl.ds(i * 16, 16)] = zv
        return carry

    lax.fori_loop(0, N_PAD // 16, zbody, 0)
    ones = jnp.ones((16,), jnp.float32)

    def body(i, carry):
        plsc.addupdate_scatter(do_v, [src_v[pl.ds(i * 16, 16)]], ones)
        plsc.addupdate_scatter(di_v, [dst_v[pl.ds(i * 16, 16)]], ones)
        return carry

    lax.fori_loop(0, EPT // 16, body, 0)
    pltpu.sync_copy(do_v, do_hbm.at[w])
    pltpu.sync_copy(di_v, di_hbm.at[w])


@functools.partial(
    pl.kernel, mesh=_mesh, compiler_params=_sc_params,
    out_type=jax.ShapeDtypeStruct((NC, N_PAD, FDIM), jnp.float32),
    scratch_types=[pltpu.VMEM((CHUNK,), jnp.int32),
                   pltpu.VMEM((CHUNK,), jnp.int32),
                   pltpu.VMEM((CHUNK, FDIM), jnp.float32),
                   pltpu.VMEM_SHARED((N_PAD, FDIM), jnp.float32),
                   pltpu.SemaphoreType.DMA],
)
def _spmm_kernel(table_hbm, src_hbm, dst_hbm, out_hbm,
                 sidx_v, didx_v, rows_v, acc_sh, sem):
    c = lax.axis_index("c")
    s = lax.axis_index("s")
    w = s * NC + c
    zv = jnp.zeros((16,), jnp.float32)

    def zrow(i, carry):
        for j in range(VECS_PER_ROW):
            rows_v[i, pl.ds(j * 16, 16)] = zv
        return carry

    lax.fori_loop(0, CHUNK, zrow, 0)

    def zacc(j, carry):
        pltpu.sync_copy(rows_v, acc_sh.at[pl.ds(s * RPT + j * CHUNK, CHUNK)])
        return carry

    lax.fori_loop(0, RPT // CHUNK, zacc, 0)
    plsc.subcore_barrier()

    base = w * EPT

    def body(i, carry):
        off = base + i * CHUNK
        pltpu.sync_copy(src_hbm.at[pl.ds(off, CHUNK)], sidx_v)
        pltpu.sync_copy(dst_hbm.at[pl.ds(off, CHUNK)], didx_v)
        pltpu.async_copy(table_hbm.at[sidx_v], rows_v, sem).wait()
        pltpu.sync_copy(rows_v, acc_sh.at[didx_v], add=True)
        return carry

    lax.fori_loop(0, NCHUNK, body, 0)
    plsc.subcore_barrier()

    def rb(j, carry):
        pltpu.sync_copy(acc_sh.at[pl.ds(s * RPT + j * CHUNK, CHUNK)], rows_v)
        pltpu.sync_copy(rows_v, out_hbm.at[c, pl.ds(s * RPT + j * CHUNK, CHUNK)])
        return carry

    lax.fori_loop(0, RPT // CHUNK, rb, 0)


def _prep_body(x_ref, g_ref, b_ref, dot_ref, dit_ref, xs_ref, m_ref, rf_ref):
    xx = x_ref[...]
    ssum = jnp.sum(xx, axis=0)
    ssq = jnp.sum(xx * xx, axis=0)
    mean = ssum / N_NODES
    var = ssq / N_NODES - mean * mean
    inv = lax.rsqrt(var + 1e-5)
    xb = (xx - mean[None, :]) * (inv * g_ref[0])[None, :] + b_ref[0][None, :]
    deg_o = jnp.sum(dot_ref[...], axis=1, keepdims=True)
    deg_i = jnp.sum(dit_ref[...], axis=1, keepdims=True)
    r_o = jnp.where(deg_o > 0, lax.rsqrt(deg_o), 0.0)
    r_i = jnp.where(deg_i > 0, lax.rsqrt(deg_i), 0.0)
    xs_ref[...] = xb * r_o
    m_ref[...] = r_i * r_o
    rf_ref[...] = r_i


_prep_call = pl.pallas_call(
    _prep_body,
    out_shape=[jax.ShapeDtypeStruct((N_PAD, FDIM), jnp.float32),
               jax.ShapeDtypeStruct((N_PAD, 1), jnp.float32),
               jax.ShapeDtypeStruct((N_PAD, 1), jnp.float32)],
)


def _combine_body(pa_ref, pb_ref, s_ref, o_ref):
    o_ref[...] = (pa_ref[0] + pb_ref[0]) * s_ref[...]


_CBLK = 1024
_combine_call = pl.pallas_call(
    _combine_body,
    grid=(N_PAD // _CBLK,),
    in_specs=[pl.BlockSpec((1, _CBLK, FDIM), lambda i: (0, i, 0)),
              pl.BlockSpec((1, _CBLK, FDIM), lambda i: (1, i, 0)),
              pl.BlockSpec((_CBLK, 1), lambda i: (i, 0))],
    out_specs=pl.BlockSpec((_CBLK, FDIM), lambda i: (i, 0)),
    out_shape=jax.ShapeDtypeStruct((N_PAD, FDIM), jnp.float32),
)


def _final_body(pa_ref, pb_ref, rf_ref, w1_ref, b1_ref, w2_ref, b2_ref,
                w3_ref, b3_ref, a_ref, o_ref):
    xp = (pa_ref[0] + pb_ref[0]) * rf_ref[...]
    a = a_ref[0, 0]
    h1 = jnp.dot(xp, w1_ref[...], preferred_element_type=jnp.float32) + b1_ref[0]
    h1 = jnp.where(h1 >= 0, h1, a * h1)
    h2 = jnp.dot(h1, w2_ref[...], preferred_element_type=jnp.float32) + b2_ref[0]
    h2 = jnp.where(h2 >= 0, h2, a * h2)
    o = jnp.dot(h2, w3_ref[...], preferred_element_type=jnp.float32) + b3_ref[0]
    m = jnp.max(o, axis=1, keepdims=True)
    lse = jnp.log(jnp.sum(jnp.exp(o - m), axis=1, keepdims=True)) + m
    o_ref[...] = o - lse


_FBLK = 1024
_final_call = pl.pallas_call(
    _final_body,
    grid=(N_PAD // _FBLK,),
    in_specs=[pl.BlockSpec((1, _FBLK, FDIM), lambda i: (0, i, 0)),
              pl.BlockSpec((1, _FBLK, FDIM), lambda i: (1, i, 0)),
              pl.BlockSpec((_FBLK, 1), lambda i: (i, 0)),
              pl.BlockSpec((FDIM, 64), lambda i: (0, 0)),
              pl.BlockSpec((1, 64), lambda i: (0, 0)),
              pl.BlockSpec((64, 64), lambda i: (0, 0)),
              pl.BlockSpec((1, 64), lambda i: (0, 0)),
              pl.BlockSpec((64, 8), lambda i: (0, 0)),
              pl.BlockSpec((1, 8), lambda i: (0, 0)),
              pl.BlockSpec((1, 1), lambda i: (0, 0))],
    out_specs=pl.BlockSpec((_FBLK, 8), lambda i: (i, 0)),
    out_shape=jax.ShapeDtypeStruct((N_PAD, 8), jnp.float32),
)


def kernel(x, x_cov, edge_index, adj_vals, gamma, beta,
           W1a, b1a, W2a, b2a, W3a,
           W1b, b1b, W2b, b2b, W3b,
           Wm1, bm1, Wm2, bm2, Wm3, bm3, prelu_a):
    src = edge_index[0].astype(jnp.int32)
    dst = edge_index[1].astype(jnp.int32)
    pad = E_PAD - N_EDGES
    padv = jnp.full((pad,), N_NODES, jnp.int32)
    src_p = jnp.concatenate([src, padv])
    dst_p = jnp.concatenate([dst, padv])
    x_pad = jnp.pad(x, ((0, N_PAD - N_NODES), (0, 0)))

    do_p, di_p = _deg_kernel(src_p, dst_p)
    xs0, mcol, rfcol = _prep_call(
        x_pad, gamma.reshape(1, FDIM), beta.reshape(1, FDIM),
        do_p.T, di_p.T)
    parts1 = _spmm_kernel(xs0, src_p, dst_p)
    y1 = _combine_call(parts1, parts1, mcol)
    parts2 = _spmm_kernel(y1, src_p, dst_p)
    out = _final_call(
        parts2, parts2, rfcol,
        Wm1, bm1.reshape(1, 64), Wm2, bm2.reshape(1, 64),
        Wm3, bm3.reshape(1, 8), prelu_a.reshape(1, 1))
    return out[:N_NODES]


# speedup vs baseline: 6.7695x; 1.1840x over previous
import functools

import jax
import jax.numpy as jnp
from jax import lax
from jax.experimental import pallas as pl
from jax.experimental.pallas import tpu as pltpu
from jax.experimental.pallas import tpu_sc as plsc

N_NODES = 10000
FDIM = 128
N_EDGES = 160000

NC = 2
NS = 16
NW = NC * NS
CHUNK = 128
EPT = 5120
E_PAD = NW * EPT
NCHUNK = EPT // CHUNK
N_PAD = 10240
RPT = N_PAD // NS
VECS_PER_ROW = FDIM // 16

_mesh = plsc.VectorSubcoreMesh(core_axis_name="c", subcore_axis_name="s")
_sc_params = pltpu.CompilerParams(needs_layout_passes=False)


@functools.partial(
    pl.kernel, mesh=_mesh, compiler_params=_sc_params,
    out_type=[jax.ShapeDtypeStruct((NW, N_PAD), jnp.float32),
              jax.ShapeDtypeStruct((NW, N_PAD), jnp.float32)],
    scratch_types=[pltpu.VMEM((EPT,), jnp.int32),
                   pltpu.VMEM((EPT,), jnp.int32),
                   pltpu.VMEM((N_PAD,), jnp.float32),
                   pltpu.VMEM((N_PAD,), jnp.float32)],
)
def _deg_kernel(src_hbm, dst_hbm, do_hbm, di_hbm, src_v, dst_v, do_v, di_v):
    c = lax.axis_index("c")
    s = lax.axis_index("s")
    w = s * NC + c
    base = w * EPT
    pltpu.sync_copy(src_hbm.at[pl.ds(base, EPT)], src_v)
    pltpu.sync_copy(dst_hbm.at[pl.ds(base, EPT)], dst_v)
    zv = jnp.zeros((16,), jnp.float32)

    def zbody(i, carry):
        do_v[pl.ds(i * 16, 16)] = zv
        di_v[pl.ds(i * 16, 16)] = zv
        return carry

    lax.fori_loop(0, N_PAD // 16, zbody, 0)
    ones = jnp.ones((16,), jnp.float32)

    def body(i, carry):
        plsc.addupdate_scatter(do_v, [src_v[pl.ds(i * 16, 16)]], ones)
        plsc.addupdate_scatter(di_v, [dst_v[pl.ds(i * 16, 16)]], ones)
        return carry

    lax.fori_loop(0, EPT // 16, body, 0)
    pltpu.sync_copy(do_v, do_hbm.at[w])
    pltpu.sync_copy(di_v, di_hbm.at[w])


@functools.partial(
    pl.kernel, mesh=_mesh, compiler_params=_sc_params,
    out_type=jax.ShapeDtypeStruct((NC, N_PAD, FDIM), jnp.float32),
    scratch_types=[pltpu.VMEM((NCHUNK, CHUNK), jnp.int32),
                   pltpu.VMEM((NCHUNK, CHUNK), jnp.int32),
                   pltpu.VMEM((CHUNK, FDIM), jnp.float32),
                   pltpu.VMEM((CHUNK, FDIM), jnp.float32),
                   pltpu.VMEM_SHARED((N_PAD, FDIM), jnp.float32),
                   pltpu.SemaphoreType.DMA,
                   pltpu.SemaphoreType.DMA,
                   pltpu.SemaphoreType.DMA,
                   pltpu.SemaphoreType.DMA],
)
def _spmm_kernel(table_hbm, src_hbm, dst_hbm, out_hbm,
                 sidx_v, didx_v, rows0, rows1, acc_sh,
                 gsem0, gsem1, ssem0, ssem1):
    c = lax.axis_index("c")
    s = lax.axis_index("s")
    w = s * NC + c
    rows = (rows0, rows1)
    gsem = (gsem0, gsem1)
    ssem = (ssem0, ssem1)

    pltpu.sync_copy(src_hbm.at[w], sidx_v)
    pltpu.sync_copy(dst_hbm.at[w], didx_v)

    zv = jnp.zeros((16,), jnp.float32)

    def zrow(i, carry):
        for j in range(VECS_PER_ROW):
            rows0[i, pl.ds(j * 16, 16)] = zv
        return carry

    lax.fori_loop(0, CHUNK, zrow, 0)

    def zacc(j, carry):
        pltpu.sync_copy(rows0, acc_sh.at[pl.ds(s * RPT + j * CHUNK, CHUNK)])
        return carry

    lax.fori_loop(0, RPT // CHUNK, zacc, 0)
    plsc.subcore_barrier()

    pltpu.async_copy(table_hbm.at[sidx_v.at[0]], rows0, gsem0)

    @pl.loop(0, NCHUNK // 2)
    def grp(g):
        for b in (0, 1):
            i = 2 * g + b
            nb = 1 - b
            pltpu.make_async_copy(
                table_hbm.at[sidx_v.at[i]], rows[b], gsem[b]).wait()
            if b == 0:
                @pl.when(g > 0)
                def _():
                    pltpu.make_async_copy(
                        rows[nb], acc_sh.at[didx_v.at[i]], ssem[nb]).wait()
            else:
                pltpu.make_async_copy(
                    rows[nb], acc_sh.at[didx_v.at[i]], ssem[nb]).wait()
            if b == 0:
                pltpu.async_copy(
                    table_hbm.at[sidx_v.at[i + 1]], rows[nb], gsem[nb])
            else:
                @pl.when(g < NCHUNK // 2 - 1)
                def _():
                    pltpu.async_copy(
                        table_hbm.at[sidx_v.at[i + 1]], rows[nb], gsem[nb])
            pltpu.async_copy(rows[b], acc_sh.at[didx_v.at[i]], ssem[b],
                             add=True)

    pltpu.make_async_copy(rows1, acc_sh.at[didx_v.at[NCHUNK - 1]], ssem1).wait()
    plsc.subcore_barrier()

    def rb(j, carry):
        pltpu.sync_copy(acc_sh.at[pl.ds(s * RPT + j * CHUNK, CHUNK)], rows0)
        pltpu.sync_copy(rows0, out_hbm.at[c, pl.ds(s * RPT + j * CHUNK, CHUNK)])
        return carry

    lax.fori_loop(0, RPT // CHUNK, rb, 0)


def _prep_body(x_ref, g_ref, b_ref, dot_ref, dit_ref, xs_ref, m_ref, rf_ref):
    xx = x_ref[...]
    ssum = jnp.sum(xx, axis=0)
    ssq = jnp.sum(xx * xx, axis=0)
    mean = ssum / N_NODES
    var = ssq / N_NODES - mean * mean
    inv = lax.rsqrt(var + 1e-5)
    xb = (xx - mean[None, :]) * (inv * g_ref[0])[None, :] + b_ref[0][None, :]
    deg_o = jnp.sum(dot_ref[...], axis=1, keepdims=True)
    deg_i = jnp.sum(dit_ref[...], axis=1, keepdims=True)
    r_o = jnp.where(deg_o > 0, lax.rsqrt(deg_o), 0.0)
    r_i = jnp.where(deg_i > 0, lax.rsqrt(deg_i), 0.0)
    xs_ref[...] = xb * r_o
    m_ref[...] = r_i * r_o
    rf_ref[...] = r_i


_prep_call = pl.pallas_call(
    _prep_body,
    out_shape=[jax.ShapeDtypeStruct((N_PAD, FDIM), jnp.float32),
               jax.ShapeDtypeStruct((N_PAD, 1), jnp.float32),
               jax.ShapeDtypeStruct((N_PAD, 1), jnp.float32)],
)


def _combine_body(pa_ref, pb_ref, s_ref, o_ref):
    o_ref[...] = (pa_ref[0] + pb_ref[0]) * s_ref[...]


_CBLK = 1024
_combine_call = pl.pallas_call(
    _combine_body,
    grid=(N_PAD // _CBLK,),
    in_specs=[pl.BlockSpec((1, _CBLK, FDIM), lambda i: (0, i, 0)),
              pl.BlockSpec((1, _CBLK, FDIM), lambda i: (1, i, 0)),
              pl.BlockSpec((_CBLK, 1), lambda i: (i, 0))],
    out_specs=pl.BlockSpec((_CBLK, FDIM), lambda i: (i, 0)),
    out_shape=jax.ShapeDtypeStruct((N_PAD, FDIM), jnp.float32),
)


def _final_body(pa_ref, pb_ref, rf_ref, w1_ref, b1_ref, w2_ref, b2_ref,
                w3_ref, b3_ref, a_ref, o_ref):
    xp = (pa_ref[0] + pb_ref[0]) * rf_ref[...]
    a = a_ref[0, 0]
    h1 = jnp.dot(xp, w1_ref[...], preferred_element_type=jnp.float32) + b1_ref[0]
    h1 = jnp.where(h1 >= 0, h1, a * h1)
    h2 = jnp.dot(h1, w2_ref[...], preferred_element_type=jnp.float32) + b2_ref[0]
    h2 = jnp.where(h2 >= 0, h2, a * h2)
    o = jnp.dot(h2, w3_ref[...], preferred_element_type=jnp.float32) + b3_ref[0]
    m = jnp.max(o, axis=1, keepdims=True)
    lse = jnp.log(jnp.sum(jnp.exp(o - m), axis=1, keepdims=True)) + m
    o_ref[...] = o - lse


_FBLK = 1024
_final_call = pl.pallas_call(
    _final_body,
    grid=(N_PAD // _FBLK,),
    in_specs=[pl.BlockSpec((1, _FBLK, FDIM), lambda i: (0, i, 0)),
              pl.BlockSpec((1, _FBLK, FDIM), lambda i: (1, i, 0)),
              pl.BlockSpec((_FBLK, 1), lambda i: (i, 0)),
              pl.BlockSpec((FDIM, 64), lambda i: (0, 0)),
              pl.BlockSpec((1, 64), lambda i: (0, 0)),
              pl.BlockSpec((64, 64), lambda i: (0, 0)),
              pl.BlockSpec((1, 64), lambda i: (0, 0)),
              pl.BlockSpec((64, 8), lambda i: (0, 0)),
              pl.BlockSpec((1, 8), lambda i: (0, 0)),
              pl.BlockSpec((1, 1), lambda i: (0, 0))],
    out_specs=pl.BlockSpec((_FBLK, 8), lambda i: (i, 0)),
    out_shape=jax.ShapeDtypeStruct((N_PAD, 8), jnp.float32),
)


def kernel(x, x_cov, edge_index, adj_vals, gamma, beta,
           W1a, b1a, W2a, b2a, W3a,
           W1b, b1b, W2b, b2b, W3b,
           Wm1, bm1, Wm2, bm2, Wm3, bm3, prelu_a):
    src = edge_index[0].astype(jnp.int32)
    dst = edge_index[1].astype(jnp.int32)
    pad = E_PAD - N_EDGES
    padv = jnp.full((pad,), N_NODES, jnp.int32)
    src_p = jnp.concatenate([src, padv])
    dst_p = jnp.concatenate([dst, padv])
    src_c = src_p.reshape(NW, NCHUNK, CHUNK)
    dst_c = dst_p.reshape(NW, NCHUNK, CHUNK)
    x_pad = jnp.pad(x, ((0, N_PAD - N_NODES), (0, 0)))

    do_p, di_p = _deg_kernel(src_p, dst_p)
    xs0, mcol, rfcol = _prep_call(
        x_pad, gamma.reshape(1, FDIM), beta.reshape(1, FDIM),
        do_p.T, di_p.T)
    parts1 = _spmm_kernel(xs0, src_c, dst_c)
    y1 = _combine_call(parts1, parts1, mcol)
    parts2 = _spmm_kernel(y1, src_c, dst_c)
    out = _final_call(
        parts2, parts2, rfcol,
        Wm1, bm1.reshape(1, 64), Wm2, bm2.reshape(1, 64),
        Wm3, bm3.reshape(1, 8), prelu_a.reshape(1, 1))
    return out[:N_NODES]


# speedup vs baseline: 9.2694x; 1.3693x over previous
import functools

import jax
import jax.numpy as jnp
from jax import lax
from jax.experimental import pallas as pl
from jax.experimental.pallas import tpu as pltpu
from jax.experimental.pallas import tpu_sc as plsc

N_NODES = 10000
FDIM = 128
N_EDGES = 160000

NC = 2
NS = 16
NW = NC * NS
CHUNK = 128
EPT = 5120
E_PAD = NW * EPT
NCHUNK = EPT // CHUNK
N_PAD = 10240
RPT = N_PAD // NS
HDIM = 64
VECS_PER_ROW = HDIM // 16

_mesh = plsc.VectorSubcoreMesh(core_axis_name="c", subcore_axis_name="s")
_sc_params = pltpu.CompilerParams(needs_layout_passes=False)
_sc_params_flat = pltpu.CompilerParams(needs_layout_passes=False,
                                       use_tc_tiling_on_sc=False)


@functools.partial(
    pl.kernel, mesh=_mesh, compiler_params=_sc_params,
    out_type=[jax.ShapeDtypeStruct((NW, N_PAD), jnp.float32),
              jax.ShapeDtypeStruct((NW, N_PAD), jnp.float32)],
    scratch_types=[pltpu.VMEM((EPT,), jnp.int32),
                   pltpu.VMEM((EPT,), jnp.int32),
                   pltpu.VMEM((N_PAD,), jnp.float32),
                   pltpu.VMEM((N_PAD,), jnp.float32)],
)
def _deg_kernel(src_hbm, dst_hbm, do_hbm, di_hbm, src_v, dst_v, do_v, di_v):
    c = lax.axis_index("c")
    s = lax.axis_index("s")
    w = s * NC + c
    base = w * EPT
    pltpu.sync_copy(src_hbm.at[pl.ds(base, EPT)], src_v)
    pltpu.sync_copy(dst_hbm.at[pl.ds(base, EPT)], dst_v)
    zv = jnp.zeros((16,), jnp.float32)

    def zbody(i, carry):
        do_v[pl.ds(i * 16, 16)] = zv
        di_v[pl.ds(i * 16, 16)] = zv
        return carry

    lax.fori_loop(0, N_PAD // 16, zbody, 0)
    ones = jnp.ones((16,), jnp.float32)

    def body(i, carry):
        plsc.addupdate_scatter(do_v, [src_v[pl.ds(i * 16, 16)]], ones)
        plsc.addupdate_scatter(di_v, [dst_v[pl.ds(i * 16, 16)]], ones)
        return carry

    lax.fori_loop(0, EPT // 16, body, 0)
    pltpu.sync_copy(do_v, do_hbm.at[w])
    pltpu.sync_copy(di_v, di_hbm.at[w])


@functools.partial(
    pl.kernel, mesh=_mesh, compiler_params=_sc_params_flat,
    out_type=jax.ShapeDtypeStruct((NC, N_PAD, HDIM), jnp.float32),
    scratch_types=[pltpu.VMEM((NCHUNK, CHUNK), jnp.int32),
                   pltpu.VMEM((NCHUNK, CHUNK), jnp.int32),
                   pltpu.VMEM((CHUNK, HDIM), jnp.float32),
                   pltpu.VMEM((CHUNK, HDIM), jnp.float32),
                   pltpu.VMEM_SHARED((N_PAD, HDIM), jnp.float32),
                   pltpu.SemaphoreType.DMA,
                   pltpu.SemaphoreType.DMA,
                   pltpu.SemaphoreType.DMA,
                   pltpu.SemaphoreType.DMA],
)
def _spmm_kernel(table_hbm, src_hbm, dst_hbm, out_hbm,
                 sidx_v, didx_v, rows0, rows1, acc_sh,
                 gsem0, gsem1, ssem0, ssem1):
    c = lax.axis_index("c")
    s = lax.axis_index("s")
    w = s * NC + c
    rows = (rows0, rows1)
    gsem = (gsem0, gsem1)
    ssem = (ssem0, ssem1)

    pltpu.sync_copy(src_hbm.at[w], sidx_v)
    pltpu.sync_copy(dst_hbm.at[w], didx_v)

    zv = jnp.zeros((16,), jnp.float32)

    def zrow(i, carry):
        for j in range(VECS_PER_ROW):
            rows0[i, pl.ds(j * 16, 16)] = zv
        return carry

    lax.fori_loop(0, CHUNK, zrow, 0)

    def zacc(j, carry):
        pltpu.sync_copy(rows0, acc_sh.at[pl.ds(s * RPT + j * CHUNK, CHUNK)])
        return carry

    lax.fori_loop(0, RPT // CHUNK, zacc, 0)
    plsc.subcore_barrier()

    pltpu.async_copy(table_hbm.at[sidx_v.at[0]], rows0, gsem0)

    @pl.loop(0, NCHUNK // 2)
    def grp(g):
        for b in (0, 1):
            i = 2 * g + b
            nb = 1 - b
            pltpu.make_async_copy(
                table_hbm.at[sidx_v.at[i]], rows[b], gsem[b]).wait()
            if b == 0:
                @pl.when(g > 0)
                def _():
                    pltpu.make_async_copy(
                        rows[nb], acc_sh.at[didx_v.at[i]], ssem[nb]).wait()
            else:
                pltpu.make_async_copy(
                    rows[nb], acc_sh.at[didx_v.at[i]], ssem[nb]).wait()
            if b == 0:
                pltpu.async_copy(
                    table_hbm.at[sidx_v.at[i + 1]], rows[nb], gsem[nb])
            else:
                @pl.when(g < NCHUNK // 2 - 1)
                def _():
                    pltpu.async_copy(
                        table_hbm.at[sidx_v.at[i + 1]], rows[nb], gsem[nb])
            pltpu.async_copy(rows[b], acc_sh.at[didx_v.at[i]], ssem[b],
                             add=True)

    pltpu.make_async_copy(rows1, acc_sh.at[didx_v.at[NCHUNK - 1]], ssem1).wait()
    plsc.subcore_barrier()

    def rb(j, carry):
        pltpu.sync_copy(acc_sh.at[pl.ds(s * RPT + j * CHUNK, CHUNK)], rows0)
        pltpu.sync_copy(rows0, out_hbm.at[c, pl.ds(s * RPT + j * CHUNK, CHUNK)])
        return carry

    lax.fori_loop(0, RPT // CHUNK, rb, 0)


def _prep_body(x_ref, g_ref, b_ref, w1_ref, dot_ref, dit_ref,
               xs_ref, m_ref, rf_ref):
    xx = x_ref[...]
    ssum = jnp.sum(xx, axis=0)
    ssq = jnp.sum(xx * xx, axis=0)
    mean = ssum / N_NODES
    var = ssq / N_NODES - mean * mean
    inv = lax.rsqrt(var + 1e-5)
    xb = (xx - mean[None, :]) * (inv * g_ref[0])[None, :] + b_ref[0][None, :]
    deg_o = jnp.sum(dot_ref[...], axis=1, keepdims=True)
    deg_i = jnp.sum(dit_ref[...], axis=1, keepdims=True)
    r_o = jnp.where(deg_o > 0, lax.rsqrt(deg_o), 0.0)
    r_i = jnp.where(deg_i > 0, lax.rsqrt(deg_i), 0.0)
    xs_ref[...] = jnp.dot(xb * r_o, w1_ref[...],
                          preferred_element_type=jnp.float32)
    m_ref[...] = r_i * r_o
    rf_ref[...] = r_i


_prep_call = pl.pallas_call(
    _prep_body,
    out_shape=[jax.ShapeDtypeStruct((N_PAD, HDIM), jnp.float32),
               jax.ShapeDtypeStruct((N_PAD, 1), jnp.float32),
               jax.ShapeDtypeStruct((N_PAD, 1), jnp.float32)],
)


def _combine_body(pa_ref, pb_ref, s_ref, o_ref):
    o_ref[...] = (pa_ref[0] + pb_ref[0]) * s_ref[...]


_CBLK = 1024
_combine_call = pl.pallas_call(
    _combine_body,
    grid=(N_PAD // _CBLK,),
    in_specs=[pl.BlockSpec((1, _CBLK, HDIM), lambda i: (0, i, 0)),
              pl.BlockSpec((1, _CBLK, HDIM), lambda i: (1, i, 0)),
              pl.BlockSpec((_CBLK, 1), lambda i: (i, 0))],
    out_specs=pl.BlockSpec((_CBLK, HDIM), lambda i: (i, 0)),
    out_shape=jax.ShapeDtypeStruct((N_PAD, HDIM), jnp.float32),
)


def _final_body(pa_ref, pb_ref, rf_ref, b1_ref, w2_ref, b2_ref,
                w3_ref, b3_ref, a_ref, o_ref):
    xp1 = (pa_ref[0] + pb_ref[0]) * rf_ref[...]
    a = a_ref[0, 0]
    h1 = xp1 + b1_ref[0]
    h1 = jnp.where(h1 >= 0, h1, a * h1)
    h2 = jnp.dot(h1, w2_ref[...], preferred_element_type=jnp.float32) + b2_ref[0]
    h2 = jnp.where(h2 >= 0, h2, a * h2)
    o = jnp.dot(h2, w3_ref[...], preferred_element_type=jnp.float32) + b3_ref[0]
    m = jnp.max(o, axis=1, keepdims=True)
    lse = jnp.log(jnp.sum(jnp.exp(o - m), axis=1, keepdims=True)) + m
    o_ref[...] = o - lse


_FBLK = 1024
_final_call = pl.pallas_call(
    _final_body,
    grid=(N_PAD // _FBLK,),
    in_specs=[pl.BlockSpec((1, _FBLK, HDIM), lambda i: (0, i, 0)),
              pl.BlockSpec((1, _FBLK, HDIM), lambda i: (1, i, 0)),
              pl.BlockSpec((_FBLK, 1), lambda i: (i, 0)),
              pl.BlockSpec((1, 64), lambda i: (0, 0)),
              pl.BlockSpec((64, 64), lambda i: (0, 0)),
              pl.BlockSpec((1, 64), lambda i: (0, 0)),
              pl.BlockSpec((64, 8), lambda i: (0, 0)),
              pl.BlockSpec((1, 8), lambda i: (0, 0)),
              pl.BlockSpec((1, 1), lambda i: (0, 0))],
    out_specs=pl.BlockSpec((_FBLK, 8), lambda i: (i, 0)),
    out_shape=jax.ShapeDtypeStruct((N_PAD, 8), jnp.float32),
)


def kernel(x, x_cov, edge_index, adj_vals, gamma, beta,
           W1a, b1a, W2a, b2a, W3a,
           W1b, b1b, W2b, b2b, W3b,
           Wm1, bm1, Wm2, bm2, Wm3, bm3, prelu_a):
    src = edge_index[0].astype(jnp.int32)
    dst = edge_index[1].astype(jnp.int32)
    pad = E_PAD - N_EDGES
    padv = jnp.full((pad,), N_NODES, jnp.int32)
    src_p = jnp.concatenate([src, padv])
    dst_p = jnp.concatenate([dst, padv])
    src_c = src_p.reshape(NW, NCHUNK, CHUNK)
    dst_c = dst_p.reshape(NW, NCHUNK, CHUNK)
    x_pad = jnp.pad(x, ((0, N_PAD - N_NODES), (0, 0)))

    do_p, di_p = _deg_kernel(src_p, dst_p)
    xs0, mcol, rfcol = _prep_call(
        x_pad, gamma.reshape(1, FDIM), beta.reshape(1, FDIM), Wm1,
        do_p.T, di_p.T)
    parts1 = _spmm_kernel(xs0, src_c, dst_c)
    y1 = _combine_call(parts1, parts1, mcol)
    parts2 = _spmm_kernel(y1, src_c, dst_c)
    out = _final_call(
        parts2, parts2, rfcol,
        bm1.reshape(1, 64), Wm2, bm2.reshape(1, 64),
        Wm3, bm3.reshape(1, 8), prelu_a.reshape(1, 1))
    return out[:N_NODES]
